# baseline (speedup 1.0000x reference)
import functools

import jax
import jax.numpy as jnp
import numpy as np
from jax import lax
from jax.experimental import pallas as pl
from jax.experimental.pallas import tpu as pltpu
from jax.experimental.pallas import tpu_sc as plsc

_B, _S, _H, _K = 4, 4096, 1024, 1024

_INTERPRET = False

_BS = 1024


def _h_body(x_ref, w1_ref, b1_ref, out_ref, chain):
    x = x_ref[0].astype(jnp.bfloat16)
    w1 = w1_ref[...].astype(jnp.bfloat16)
    if chain == 'seq256':
        h = None
        for c in range(_H // 256):
            p = lax.dot_general(x[:, c * 256:(c + 1) * 256],
                                w1[:, c * 256:(c + 1) * 256],
                                (((1,), (1,)), ((), ())),
                                preferred_element_type=jnp.float32)
            h = p if h is None else h + p
    else:
        h = lax.dot_general(x, w1, (((1,), (1,)), ((), ())),
                            preferred_element_type=jnp.float32)
    out_ref[0] = h + b1_ref[...]


def _h_two(x2, W1, b1, chain):
    return pl.pallas_call(
        functools.partial(_h_body, chain=chain),
        grid=(2, _S // _BS),
        in_specs=[
            pl.BlockSpec((1, _BS, _H), lambda b, s: (b, s, 0)),
            pl.BlockSpec((_H, _H), lambda b, s: (0, 0)),
            pl.BlockSpec((1, _H), lambda b, s: (0, 0)),
        ],
        out_specs=pl.BlockSpec((1, _BS, _H), lambda b, s: (b, s, 0)),
        out_shape=jax.ShapeDtypeStruct((2, _S, _H), jnp.float32),
        interpret=_INTERPRET,
    )(x2, W1, b1.reshape(1, _H))


def _mlp_h(x, W1, b1):
    h02 = _h_two(x[0::2], W1, b1, 'seq256')
    h13 = _h_two(x[1::2], W1, b1, 'single')
    return jnp.stack([h02, h13], axis=1).reshape(_B, _S, _H)


def _mv_body(g_ref, w2_ref, b2_ref, out_ref):
    g = g_ref[0].astype(jnp.bfloat16)
    w = w2_ref[...].astype(jnp.bfloat16)
    lg = lax.dot_general(g, w, (((1,), (1,)), ((), ())),
                         preferred_element_type=jnp.float32)
    out_ref[0] = lg[:, 0:1] + b2_ref[0, 0]


def _matvec(g, W2, b2):
    w2p = jnp.zeros((8, _H), jnp.float32).at[0].set(W2[0])
    return pl.pallas_call(
        _mv_body,
        grid=(_B, _S // _BS),
        in_specs=[
            pl.BlockSpec((1, _BS, _H), lambda b, s: (b, s, 0)),
            pl.BlockSpec((8, _H), lambda b, s: (0, 0)),
            pl.BlockSpec((1, 1), lambda b, s: (0, 0)),
        ],
        out_specs=pl.BlockSpec((1, _BS, 1), lambda b, s: (b, s, 0)),
        out_shape=jax.ShapeDtypeStruct((_B, _S, 1), jnp.float32),
        interpret=_INTERPRET,
    )(g, w2p, b2.reshape(1, 1))


_IB = 512
_JC = 1024


def _rank_body(a_ref, l_ref, out_ref):
    a = a_ref[0]
    i0 = pl.program_id(1) * _IB
    i_idx = lax.broadcasted_iota(jnp.int32, (_IB, 1), 0) + i0
    cnt = jnp.zeros((_IB, 1), jnp.float32)
    for c in range(_S // _JC):
        lj = l_ref[0, :, c * _JC:(c + 1) * _JC]
        gt = lj > a
        eq = lj == a
        j_idx = lax.broadcasted_iota(jnp.int32, (_IB, _JC), 1) + c * _JC
        pred = gt | (eq & (j_idx < i_idx))
        cnt = cnt + jnp.sum(pred.astype(jnp.float32), axis=1, keepdims=True)
    out_ref[0] = cnt


def _ranks(logits_col, logits_row):
    return pl.pallas_call(
        _rank_body,
        grid=(_B, _S // _IB),
        in_specs=[
            pl.BlockSpec((1, _IB, 1), lambda b, i: (b, i, 0)),
            pl.BlockSpec((1, 1, _S), lambda b, i: (b, 0, 0)),
        ],
        out_specs=pl.BlockSpec((1, _IB, 1), lambda b, i: (b, i, 0)),
        out_shape=jax.ShapeDtypeStruct((_B, _S, 1), jnp.float32),
        interpret=_INTERPRET,
    )(logits_col, logits_row)


_KB = 512


def _extract_body(rk_ref, l_ref, out_ref):
    r_col = rk_ref[0]
    b = pl.program_id(0)
    rbase = pl.program_id(1) * _KB
    rvals = (lax.broadcasted_iota(jnp.int32, (_S, _KB), 1) + rbase).astype(jnp.float32)
    onehot = (r_col == rvals).astype(jnp.float32)
    tok = (lax.broadcasted_iota(jnp.int32, (1, _S), 1) + b * _S).astype(jnp.float32)
    A = jnp.concatenate([tok, l_ref[0]], axis=0)
    res = lax.dot_general(A, onehot, (((1,), (0,)), ((), ())),
                          precision=lax.Precision.HIGHEST,
                          preferred_element_type=jnp.float32)
    out_ref[0] = res


def _extract(ranks_col, logits_row):
    return pl.pallas_call(
        _extract_body,
        grid=(_B, _K // _KB),
        in_specs=[
            pl.BlockSpec((1, _S, 1), lambda b, r: (b, 0, 0)),
            pl.BlockSpec((1, 1, _S), lambda b, r: (b, 0, 0)),
        ],
        out_specs=pl.BlockSpec((1, 2, _KB), lambda b, r: (b, 0, r)),
        out_shape=jax.ShapeDtypeStruct((_B, 2, _K), jnp.float32),
        interpret=_INTERPRET,
    )(ranks_col, logits_row)


_NC, _NS = 2, 16
_NW = _NC * _NS
_RPW = _B * _K // _NW
_CH = 32


def _sc_gather_body(feat_hbm, sel_hbm, w_hbm, out_hbm, idx_v, w_v, rows_v, sem):
    c = lax.axis_index("c")
    s = lax.axis_index("s")
    wid = s * _NC + c
    base = wid * _RPW
    maxidx = _B * _S - 1
    for ci in range(_RPW // _CH):
        r0 = base + ci * _CH
        pltpu.sync_copy(sel_hbm.at[pl.ds(r0, _CH)], idx_v)
        pltpu.sync_copy(w_hbm.at[pl.ds(r0, _CH)], w_v)
        for t in range(_CH // 16):
            sl = pl.ds(t * 16, 16)
            idx_v[sl] = jnp.minimum(jnp.maximum(idx_v[sl], 0), maxidx)
        pltpu.async_copy(feat_hbm.at[idx_v], rows_v, sem).wait()

        def scale_row(r, carry):
            wv = w_v[r]
            for hh in range(_H // 16):
                sl = pl.ds(hh * 16, 16)
                rows_v[r, sl] = rows_v[r, sl] * wv
            return carry

        lax.fori_loop(0, _CH, scale_row, 0)
        pltpu.sync_copy(rows_v, out_hbm.at[pl.ds(r0, _CH)])


def _sc_gather(feat_flat, sel_flat, w_flat):
    mesh = plsc.VectorSubcoreMesh(core_axis_name="c", subcore_axis_name="s")
    w_flat = jnp.broadcast_to(w_flat[:, None], (_B * _K, 16))
    fn = functools.partial(
        pl.kernel,
        out_type=jax.ShapeDtypeStruct((_B * _K, _H), jnp.float32),
        mesh=mesh,
        scratch_types=[
            pltpu.VMEM((_CH,), jnp.int32),
            pltpu.VMEM((_CH, 16), jnp.float32),
            pltpu.VMEM((_CH, _H), jnp.float32),
            pltpu.SemaphoreType.DMA,
        ],
    )(_sc_gather_body)
    return fn(feat_flat, sel_flat, w_flat)


def _logits_all(image_features, W1, b1, W2, b2):
    h = _mlp_h(image_features, W1, b1)
    g = jax.nn.gelu(h, approximate=False)
    return _matvec(g, W2, b2)


def kernel(image_features, W1, b1, W2, b2):
    logits_col = _logits_all(image_features, W1, b1, W2, b2)
    logits_row = logits_col.reshape(_B, 1, _S)
    ranks_col = _ranks(logits_col, logits_row)
    selw = _extract(ranks_col, logits_row)
    sel = selw[:, 0, :].reshape(_B * _K).astype(jnp.int32)
    w = selw[:, 1, :].reshape(_B * _K)
    feat_flat = image_features.reshape(_B * _S, _H)
    out = _sc_gather(feat_flat, sel, w)
    return out.reshape(_B, _K, _H)


# speedup vs baseline: 1.3014x; 1.3014x over previous
import functools

import jax
import jax.numpy as jnp
import numpy as np
from jax import lax
from jax.experimental import pallas as pl
from jax.experimental.pallas import tpu as pltpu
from jax.experimental.pallas import tpu_sc as plsc

_B, _S, _H, _K = 4, 4096, 1024, 1024

_INTERPRET = False

_BS = 1024


def _h_body(x_ref, w1_ref, b1_ref, out_ref, chain):
    x = x_ref[0].astype(jnp.bfloat16)
    w1 = w1_ref[...].astype(jnp.bfloat16)
    if chain == 'seq256':
        h = None
        for c in range(_H // 256):
            p = lax.dot_general(x[:, c * 256:(c + 1) * 256],
                                w1[:, c * 256:(c + 1) * 256],
                                (((1,), (1,)), ((), ())),
                                preferred_element_type=jnp.float32)
            h = p if h is None else h + p
    else:
        h = lax.dot_general(x, w1, (((1,), (1,)), ((), ())),
                            preferred_element_type=jnp.float32)
    out_ref[0] = h + b1_ref[...]


def _h_two(x2, W1, b1, chain):
    return pl.pallas_call(
        functools.partial(_h_body, chain=chain),
        grid=(2, _S // _BS),
        in_specs=[
            pl.BlockSpec((1, _BS, _H), lambda b, s: (b, s, 0)),
            pl.BlockSpec((_H, _H), lambda b, s: (0, 0)),
            pl.BlockSpec((1, _H), lambda b, s: (0, 0)),
        ],
        out_specs=pl.BlockSpec((1, _BS, _H), lambda b, s: (b, s, 0)),
        out_shape=jax.ShapeDtypeStruct((2, _S, _H), jnp.float32),
        interpret=_INTERPRET,
    )(x2, W1, b1.reshape(1, _H))


def _mlp_h(x, W1, b1):
    h02 = _h_two(x[0::2], W1, b1, 'seq256')
    h13 = _h_two(x[1::2], W1, b1, 'single')
    return h02, h13


def _mv_body(g_ref, w2_ref, b2_ref, out_ref):
    g = g_ref[0].astype(jnp.bfloat16)
    w = w2_ref[...].astype(jnp.bfloat16)
    lg = lax.dot_general(g, w, (((1,), (1,)), ((), ())),
                         preferred_element_type=jnp.float32)
    out_ref[0] = lg[:, 0:1] + b2_ref[0, 0]


def _matvec(g, W2, b2):
    nb = g.shape[0]
    w2p = jnp.zeros((8, _H), jnp.float32).at[0].set(W2[0])
    return pl.pallas_call(
        _mv_body,
        grid=(nb, _S // _BS),
        in_specs=[
            pl.BlockSpec((1, _BS, _H), lambda b, s: (b, s, 0)),
            pl.BlockSpec((8, _H), lambda b, s: (0, 0)),
            pl.BlockSpec((1, 1), lambda b, s: (0, 0)),
        ],
        out_specs=pl.BlockSpec((1, _BS, 1), lambda b, s: (b, s, 0)),
        out_shape=jax.ShapeDtypeStruct((nb, _S, 1), jnp.float32),
        interpret=_INTERPRET,
    )(g, w2p, b2.reshape(1, 1))


_IB = 512
_JC = 1024


def _rank_body(a_ref, l_ref, out_ref):
    a = a_ref[0]
    i0 = pl.program_id(1) * _IB
    i_idx = lax.broadcasted_iota(jnp.int32, (_IB, 1), 0) + i0
    cnt = jnp.zeros((_IB, 1), jnp.float32)
    for c in range(_S // _JC):
        lj = l_ref[0, :, c * _JC:(c + 1) * _JC]
        gt = lj > a
        eq = lj == a
        j_idx = lax.broadcasted_iota(jnp.int32, (_IB, _JC), 1) + c * _JC
        pred = gt | (eq & (j_idx < i_idx))
        cnt = cnt + jnp.sum(pred.astype(jnp.float32), axis=1, keepdims=True)
    out_ref[0] = cnt


def _ranks(logits_col, logits_row):
    nb = logits_col.shape[0]
    return pl.pallas_call(
        _rank_body,
        grid=(nb, _S // _IB),
        in_specs=[
            pl.BlockSpec((1, _IB, 1), lambda b, i: (b, i, 0)),
            pl.BlockSpec((1, 1, _S), lambda b, i: (b, 0, 0)),
        ],
        out_specs=pl.BlockSpec((1, _IB, 1), lambda b, i: (b, i, 0)),
        out_shape=jax.ShapeDtypeStruct((nb, _S, 1), jnp.float32),
        interpret=_INTERPRET,
    )(logits_col, logits_row)


_KB = 512


def _extract_body(rk_ref, l_ref, out_ref, off):
    r_col = rk_ref[0]
    b = pl.program_id(0) * 2 + off
    rbase = pl.program_id(1) * _KB
    rvals = (lax.broadcasted_iota(jnp.int32, (_S, _KB), 1) + rbase).astype(jnp.float32)
    onehot = (r_col == rvals).astype(jnp.float32)
    tok = (lax.broadcasted_iota(jnp.int32, (1, _S), 1) + b * _S).astype(jnp.float32)
    A = jnp.concatenate([tok, l_ref[0]], axis=0)
    res = lax.dot_general(A, onehot, (((1,), (0,)), ((), ())),
                          precision=lax.Precision.HIGHEST,
                          preferred_element_type=jnp.float32)
    out_ref[0] = res


def _extract(ranks_col, logits_row, off):
    nb = ranks_col.shape[0]
    return pl.pallas_call(
        functools.partial(_extract_body, off=off),
        grid=(nb, _K // _KB),
        in_specs=[
            pl.BlockSpec((1, _S, 1), lambda b, r: (b, 0, 0)),
            pl.BlockSpec((1, 1, _S), lambda b, r: (b, 0, 0)),
        ],
        out_specs=pl.BlockSpec((1, 2, _KB), lambda b, r: (b, 0, r)),
        out_shape=jax.ShapeDtypeStruct((nb, 2, _K), jnp.float32),
        interpret=_INTERPRET,
    )(ranks_col, logits_row)


_NC, _NS = 2, 16
_NW = _NC * _NS
_RPW = _B * _K // _NW
_CH = 32


def _sc_gather_body(feat_hbm, sel_hbm, w_hbm, out_hbm, idx_v, w_v, rows_v, sem):
    c = lax.axis_index("c")
    s = lax.axis_index("s")
    wid = s * _NC + c
    base = wid * _RPW
    maxidx = _B * _S - 1
    for ci in range(_RPW // _CH):
        r0 = base + ci * _CH
        pltpu.sync_copy(sel_hbm.at[pl.ds(r0, _CH)], idx_v)
        pltpu.sync_copy(w_hbm.at[pl.ds(r0, _CH)], w_v)
        for t in range(_CH // 16):
            sl = pl.ds(t * 16, 16)
            idx_v[sl] = jnp.minimum(jnp.maximum(idx_v[sl], 0), maxidx)
        pltpu.async_copy(feat_hbm.at[idx_v], rows_v, sem).wait()

        def scale_row(r, carry):
            wv = w_v[r]
            for hh in range(_H // 16):
                sl = pl.ds(hh * 16, 16)
                rows_v[r, sl] = rows_v[r, sl] * wv
            return carry

        lax.fori_loop(0, _CH, scale_row, 0)
        pltpu.sync_copy(rows_v, out_hbm.at[pl.ds(r0, _CH)])


def _sc_gather(feat_flat, sel_flat, w_flat):
    mesh = plsc.VectorSubcoreMesh(core_axis_name="c", subcore_axis_name="s")
    w_flat = jnp.broadcast_to(w_flat[:, None], (_B * _K, 16))
    fn = functools.partial(
        pl.kernel,
        out_type=jax.ShapeDtypeStruct((_B * _K, _H), jnp.float32),
        mesh=mesh,
        scratch_types=[
            pltpu.VMEM((_CH,), jnp.int32),
            pltpu.VMEM((_CH, 16), jnp.float32),
            pltpu.VMEM((_CH, _H), jnp.float32),
            pltpu.SemaphoreType.DMA,
        ],
    )(_sc_gather_body)
    return fn(feat_flat, sel_flat, w_flat)


def _logits_all(image_features, W1, b1, W2, b2):
    h02, h13 = _mlp_h(image_features, W1, b1)
    outs = []
    for hh in (h02, h13):
        g = jax.nn.gelu(hh, approximate=False)
        outs.append(_matvec(g, W2, b2))
    return outs


def _selw_pair(logits_col, off):
    logits_row = logits_col.reshape(2, 1, _S)
    ranks_col = _ranks(logits_col, logits_row)
    return _extract(ranks_col, logits_row, off)


def kernel(image_features, W1, b1, W2, b2):
    lc02, lc13 = _logits_all(image_features, W1, b1, W2, b2)
    selw = jnp.stack([_selw_pair(lc02, 0), _selw_pair(lc13, 1)],
                     axis=1).reshape(_B, 2, _K)
    sel = selw[:, 0, :].reshape(_B * _K).astype(jnp.int32)
    w = selw[:, 1, :].reshape(_B * _K)
    feat_flat = image_features.reshape(_B * _S, _H)
    out = _sc_gather(feat_flat, sel, w)
    return out.reshape(_B, _K, _H)
